# baseline (speedup 1.0000x reference)
import functools
import jax
import jax.numpy as jnp
from jax import lax
from jax.experimental import pallas as pl
from jax.experimental.pallas import tpu as pltpu
from jax.experimental.pallas import tpu_sc as plsc

B = 16384
V = 11
D = 64
NC, NS = 2, 16
NW = NC * NS
BPW = B // NW
CHUNK = 128
NCH = BPW // CHUNK


@functools.cache
def _make_sc_kernel():
    @functools.partial(
        pl.kernel,
        mesh=plsc.VectorSubcoreMesh(core_axis_name="c", subcore_axis_name="s"),
        out_type=jax.ShapeDtypeStruct((B, 2 * D), jnp.float32),
        scratch_types=[
            pltpu.VMEM((V, D), jnp.float32),
            pltpu.VMEM((V, D), jnp.float32),
            pltpu.VMEM((BPW, 2 * D), jnp.float32),
            pltpu.VMEM((BPW + 16,), jnp.int32),
            pltpu.VMEM((BPW + 16,), jnp.int32),
            pltpu.SemaphoreType.DMA,
            pltpu.SemaphoreType.DMA,
        ],
    )
    def _sc_body(y_hbm, s_hbm, e1_hbm, e2_hbm, out_hbm,
                 e1_v, e2_v, rows_v, y_v, s_v, sem_in, sem_o):
        wid = lax.axis_index("s") * NC + lax.axis_index("c")
        base = wid * BPW
        loads = [
            pltpu.async_copy(e1_hbm, e1_v, sem_in),
            pltpu.async_copy(e2_hbm, e2_v, sem_in),
            pltpu.async_copy(y_hbm.at[pl.ds(base, BPW)],
                             y_v.at[pl.ds(0, BPW)], sem_in),
            pltpu.async_copy(s_hbm.at[pl.ds(base, BPW)],
                             s_v.at[pl.ds(0, BPW)], sem_in),
        ]
        for cp in loads:
            cp.wait()

        def row(i, _):
            yi = y_v[pl.ds(i, 16)][0]
            si = s_v[pl.ds(i, 16)][0]
            for m in range(D // 16):
                sl = pl.ds(m * 16, 16)
                rows_v[i, sl] = e1_v[yi, sl]
            for m in range(D // 16):
                sl = pl.ds(m * 16, 16)
                rows_v[i, pl.ds(D + m * 16, 16)] = e2_v[si, sl]
            return _

        outs = []
        for c in range(NCH):
            lax.fori_loop(c * CHUNK, (c + 1) * CHUNK, row, None,
                          unroll=4)
            outs.append(
                pltpu.async_copy(
                    rows_v.at[pl.ds(c * CHUNK, CHUNK)],
                    out_hbm.at[pl.ds(base + c * CHUNK, CHUNK)],
                    sem_o))
        for cp in outs:
            cp.wait()

    return _sc_body


def kernel(y, s, emb1, emb2):
    return _make_sc_kernel()(y.astype(jnp.int32), s.astype(jnp.int32),
                             emb1, emb2)


# speedup vs baseline: 1.0419x; 1.0419x over previous
import functools
import jax
import jax.numpy as jnp
from jax import lax
from jax.experimental import pallas as pl
from jax.experimental.pallas import tpu as pltpu
from jax.experimental.pallas import tpu_sc as plsc

B = 16384
V = 11
D = 64
W = 2 * D
NC, NS = 2, 16
NW = NC * NS
BPW = B // NW
CHUNK = 128
NCH = BPW // CHUNK
G = 16
TROWS = 128


def _table_body(e1_ref, e2_ref, out_ref):
    for a in range(V):
        out_ref[pl.ds(a * V, V), pl.ds(0, D)] = jnp.broadcast_to(
            e1_ref[pl.ds(a, 1), :], (V, D))
        out_ref[pl.ds(a * V, V), pl.ds(D, D)] = e2_ref[...]


def _build_table(emb1, emb2):
    return pl.pallas_call(
        _table_body,
        out_shape=jax.ShapeDtypeStruct((TROWS, W), jnp.float32),
    )(emb1, emb2)


@functools.cache
def _make_sc_kernel():
    @functools.partial(
        pl.kernel,
        mesh=plsc.VectorSubcoreMesh(core_axis_name="c", subcore_axis_name="s"),
        out_type=jax.ShapeDtypeStruct((B * W,), jnp.float32),
        scratch_types=[
            pltpu.VMEM((TROWS * W,), jnp.float32),
            pltpu.VMEM((BPW * W,), jnp.float32),
            pltpu.VMEM((BPW,), jnp.int32),
            pltpu.VMEM((BPW,), jnp.int32),
            pltpu.VMEM((BPW,), jnp.int32),
            pltpu.SemaphoreType.DMA,
            pltpu.SemaphoreType.DMA,
            pltpu.SemaphoreType.DMA,
        ],
    )
    def _sc_body(y_hbm, s_hbm, tab_hbm, out_hbm,
                 tab_v, rows_v, y_v, s_v, off_v, sem_t, sem_in, sem_o):
        wid = lax.axis_index("s") * NC + lax.axis_index("c")
        base = wid * BPW
        ct = pltpu.async_copy(tab_hbm, tab_v, sem_t)
        loads = [
            pltpu.async_copy(y_hbm.at[pl.ds(base, BPW)], y_v, sem_in),
            pltpu.async_copy(s_hbm.at[pl.ds(base, BPW)], s_v, sem_in),
        ]
        for cp in loads:
            cp.wait()
        for k in range(BPW // G):
            sl = pl.ds(k * G, G)
            off_v[sl] = (y_v[sl] * V + s_v[sl]) * W
        ct.wait()

        def group(g, _):
            i0 = g * G
            offs = off_v[pl.ds(i0, G)]
            for l in range(G):
                tb = offs[l]
                ob = (i0 + l) * W
                for m in range(W // 16):
                    rows_v[pl.ds(ob + m * 16, 16)] = \
                        tab_v[pl.ds(tb + m * 16, 16)]
            return _

        outs = []
        gpc = CHUNK // G
        cw = CHUNK * W
        for c in range(NCH):
            lax.fori_loop(c * gpc, (c + 1) * gpc, group, None)
            outs.append(
                pltpu.async_copy(
                    rows_v.at[pl.ds(c * cw, cw)],
                    out_hbm.at[pl.ds(base * W + c * cw, cw)],
                    sem_o))
        for cp in outs:
            cp.wait()

    return _sc_body


def kernel(y, s, emb1, emb2):
    table = _build_table(emb1, emb2)
    out = _make_sc_kernel()(y.astype(jnp.int32), s.astype(jnp.int32),
                            table.reshape(TROWS * W))
    return out.reshape(B, W)


# speedup vs baseline: 1.7683x; 1.6972x over previous
import functools
import jax
import jax.numpy as jnp
from jax import lax
from jax.experimental import pallas as pl
from jax.experimental.pallas import tpu as pltpu
from jax.experimental.pallas import tpu_sc as plsc

B = 16384
V = 11
D = 64
W = 2 * D
NC, NS = 2, 16
NW = NC * NS
BPW = B // NW
CHUNK = 128
NCH = BPW // CHUNK
TROWS = 128


def _table_body(e1_ref, e2_ref, out_ref):
    for a in range(V):
        out_ref[pl.ds(a * V, V), pl.ds(0, D)] = jnp.broadcast_to(
            e1_ref[pl.ds(a, 1), :], (V, D))
        out_ref[pl.ds(a * V, V), pl.ds(D, D)] = e2_ref[...]


def _build_table(emb1, emb2):
    return pl.pallas_call(
        _table_body,
        out_shape=jax.ShapeDtypeStruct((TROWS, W), jnp.float32),
    )(emb1, emb2)


@functools.cache
def _make_sc_gather():
    @functools.partial(
        pl.kernel,
        mesh=plsc.VectorSubcoreMesh(core_axis_name="c", subcore_axis_name="s"),
        out_type=jax.ShapeDtypeStruct((NW * NCH, CHUNK, W), jnp.float32),
        scratch_types=[
            pltpu.VMEM_SHARED((TROWS, W), jnp.float32),
            pltpu.VMEM((NCH, CHUNK), jnp.int32),
            pltpu.VMEM((NCH, CHUNK), jnp.int32),
            pltpu.VMEM((NCH, CHUNK), jnp.int32),
            pltpu.VMEM((NCH, CHUNK, W), jnp.float32),
            pltpu.SemaphoreType.DMA,
            pltpu.SemaphoreType.DMA,
        ],
    )
    def _sc_gather(y_hbm, s_hbm, tab_hbm, out_hbm,
                   tab_sh, y_v, s_v, idx_v, rows_v, sem_g, sem_in):
        sid = lax.axis_index("s")
        wid = sid * NC + lax.axis_index("c")
        base = wid * NCH
        loads = [
            pltpu.async_copy(y_hbm.at[pl.ds(base, NCH)], y_v, sem_in),
            pltpu.async_copy(s_hbm.at[pl.ds(base, NCH)], s_v, sem_in),
        ]

        @pl.when(sid == 0)
        def _stage():
            pltpu.sync_copy(tab_hbm, tab_sh)

        for cp in loads:
            cp.wait()
        for c in range(NCH):
            for m in range(CHUNK // 16):
                sl = pl.ds(m * 16, 16)
                idx_v[c, sl] = y_v[c, sl] * V + s_v[c, sl]
        plsc.subcore_barrier()
        gathers = [
            pltpu.async_copy(tab_sh.at[idx_v.at[c]], rows_v.at[c], sem_g)
            for c in range(NCH)
        ]
        for cp in gathers:
            cp.wait()
        pltpu.sync_copy(rows_v, out_hbm.at[pl.ds(base, NCH)])

    return _sc_gather


def kernel(y, s, emb1, emb2):
    table = _build_table(emb1, emb2)
    y2 = y.astype(jnp.int32).reshape(NW * NCH, CHUNK)
    s2 = s.astype(jnp.int32).reshape(NW * NCH, CHUNK)
    out = _make_sc_gather()(y2, s2, table)
    return out.reshape(B, W)
